# baseline (device time: 173915 ns/iter reference)
import jax
import jax.numpy as jnp
from jax import lax
from jax.experimental import pallas as pl
from jax.experimental.pallas import tpu as pltpu

N_DEV = 4
E_TOTAL = 16


def kernel(x, router_W, route_idx, expert_W):
    n_tok, d_model = x.shape
    e_loc, _, d_hid = expert_W.shape

    scores = x @ router_W
    probs = jax.nn.softmax(scores, axis=-1)
    oh0 = jax.nn.one_hot(route_idx[:, 0], E_TOTAL, dtype=jnp.float32)
    oh1 = jax.nn.one_hot(route_idx[:, 1], E_TOTAL, dtype=jnp.float32)
    g0 = (probs * oh0).sum(axis=-1, keepdims=True)
    g1 = (probs * oh1).sum(axis=-1, keepdims=True)
    w = (oh0 * g0 + oh1 * g1) / (g0 + g1)

    x_bf = x.astype(jnp.bfloat16)
    ew_bf = expert_W.astype(jnp.bfloat16)

    def body(x_ref, w_ref, ew_ref, out_ref, wall_ref, send_sems, recv_sems):
        my = lax.axis_index("i")
        left = lax.rem(my + N_DEV - 1, N_DEV)
        right = lax.rem(my + 1, N_DEV)

        barrier_sem = pltpu.get_barrier_semaphore()
        for nbr in (left, right):
            pl.semaphore_signal(
                barrier_sem, inc=1,
                device_id=(nbr,), device_id_type=pl.DeviceIdType.MESH,
            )
        pl.semaphore_wait(barrier_sem, 2)

        wall_ref[pl.ds(my * e_loc, e_loc)] = ew_ref[...]

        for h in range(N_DEV - 1):
            send_chunk = lax.rem(my - h + N_DEV, N_DEV)
            rdma = pltpu.make_async_remote_copy(
                src_ref=wall_ref.at[pl.ds(send_chunk * e_loc, e_loc)],
                dst_ref=wall_ref.at[pl.ds(send_chunk * e_loc, e_loc)],
                send_sem=send_sems.at[h],
                recv_sem=recv_sems.at[h],
                device_id=(right,),
                device_id_type=pl.DeviceIdType.MESH,
            )
            rdma.start()
            rdma.wait()

        xb = x_ref[...]
        acc = jnp.zeros((n_tok, d_hid), jnp.float32)
        for e in range(E_TOTAL):
            y = jnp.dot(xb, wall_ref[e], preferred_element_type=jnp.float32)
            acc = acc + w_ref[:, e : e + 1] * y
        out_ref[...] = acc

    return pl.pallas_call(
        body,
        out_shape=jax.ShapeDtypeStruct((n_tok, d_hid), jnp.float32),
        in_specs=[
            pl.BlockSpec(memory_space=pltpu.VMEM),
            pl.BlockSpec(memory_space=pltpu.VMEM),
            pl.BlockSpec(memory_space=pltpu.VMEM),
        ],
        out_specs=pl.BlockSpec(memory_space=pltpu.VMEM),
        scratch_shapes=[
            pltpu.VMEM((E_TOTAL, d_model, d_hid), jnp.bfloat16),
            pltpu.SemaphoreType.DMA((N_DEV - 1,)),
            pltpu.SemaphoreType.DMA((N_DEV - 1,)),
        ],
        compiler_params=pltpu.CompilerParams(collective_id=0),
    )(x_bf, w, ew_bf)


# device time: 88387 ns/iter; 1.9677x vs baseline; 1.9677x over previous
import jax
import jax.numpy as jnp
from jax import lax
from jax.experimental import pallas as pl
from jax.experimental.pallas import tpu as pltpu

N_DEV = 4
E_TOTAL = 16
E_LOC = 4


def kernel(x, router_W, route_idx, expert_W):
    n_tok, d_model = x.shape
    e_loc, _, d_hid = expert_W.shape

    scores = x @ router_W
    probs = jax.nn.softmax(scores, axis=-1)
    oh0 = jax.nn.one_hot(route_idx[:, 0], E_TOTAL, dtype=jnp.float32)
    oh1 = jax.nn.one_hot(route_idx[:, 1], E_TOTAL, dtype=jnp.float32)
    g0 = (probs * oh0).sum(axis=-1, keepdims=True)
    g1 = (probs * oh1).sum(axis=-1, keepdims=True)
    w = (oh0 * g0 + oh1 * g1) / (g0 + g1)

    me_out = lax.axis_index("i")
    w_rel = lax.dynamic_slice(
        jnp.concatenate([w, w], axis=1), (0, E_LOC * me_out), (n_tok, E_TOTAL)
    )

    x_bf = x.astype(jnp.bfloat16)
    ew_bf = expert_W.astype(jnp.bfloat16)

    def body(x_ref, w_ref, ew_ref, out_ref, wall_ref, send_sems, recv_sems):
        my = lax.axis_index("i")
        left = lax.rem(my + N_DEV - 1, N_DEV)
        right = lax.rem(my + 1, N_DEV)

        barrier_sem = pltpu.get_barrier_semaphore()
        for nbr in (left, right):
            pl.semaphore_signal(
                barrier_sem, inc=1,
                device_id=(nbr,), device_id_type=pl.DeviceIdType.MESH,
            )
        pl.semaphore_wait(barrier_sem, 2)

        wall_ref[pl.ds(0, E_LOC)] = ew_ref[...]

        d0R = pltpu.make_async_remote_copy(
            src_ref=wall_ref.at[pl.ds(0, E_LOC)],
            dst_ref=wall_ref.at[pl.ds(3 * E_LOC, E_LOC)],
            send_sem=send_sems.at[0], recv_sem=recv_sems.at[0],
            device_id=(right,), device_id_type=pl.DeviceIdType.MESH,
        )
        d0L = pltpu.make_async_remote_copy(
            src_ref=wall_ref.at[pl.ds(0, E_LOC)],
            dst_ref=wall_ref.at[pl.ds(1 * E_LOC, E_LOC)],
            send_sem=send_sems.at[1], recv_sem=recv_sems.at[1],
            device_id=(left,), device_id_type=pl.DeviceIdType.MESH,
        )
        d0R.start()
        d0L.start()

        xb = x_ref[...]

        def chunk_mm(slot):
            acc = jnp.zeros((n_tok, d_hid), jnp.float32)
            for j in range(E_LOC):
                e = slot * E_LOC + j
                y = jnp.dot(xb, wall_ref[e], preferred_element_type=jnp.float32)
                acc = acc + w_ref[:, e : e + 1] * y
            return acc

        acc = chunk_mm(0)

        d0R.wait_recv()
        d0L.wait_recv()

        d1R = pltpu.make_async_remote_copy(
            src_ref=wall_ref.at[pl.ds(3 * E_LOC, 2)],
            dst_ref=wall_ref.at[pl.ds(2 * E_LOC, 2)],
            send_sem=send_sems.at[2], recv_sem=recv_sems.at[2],
            device_id=(right,), device_id_type=pl.DeviceIdType.MESH,
        )
        d1L = pltpu.make_async_remote_copy(
            src_ref=wall_ref.at[pl.ds(1 * E_LOC + 2, 2)],
            dst_ref=wall_ref.at[pl.ds(2 * E_LOC + 2, 2)],
            send_sem=send_sems.at[3], recv_sem=recv_sems.at[3],
            device_id=(left,), device_id_type=pl.DeviceIdType.MESH,
        )
        d1R.start()
        d1L.start()

        acc = acc + chunk_mm(1) + chunk_mm(3)

        d1R.wait_recv()
        d1L.wait_recv()
        acc = acc + chunk_mm(2)
        out_ref[...] = acc

        d0R.wait_send()
        d0L.wait_send()
        d1R.wait_send()
        d1L.wait_send()

    return pl.pallas_call(
        body,
        out_shape=jax.ShapeDtypeStruct((n_tok, d_hid), jnp.float32),
        in_specs=[
            pl.BlockSpec(memory_space=pltpu.VMEM),
            pl.BlockSpec(memory_space=pltpu.VMEM),
            pl.BlockSpec(memory_space=pltpu.VMEM),
        ],
        out_specs=pl.BlockSpec(memory_space=pltpu.VMEM),
        scratch_shapes=[
            pltpu.VMEM((E_TOTAL, d_model, d_hid), jnp.bfloat16),
            pltpu.SemaphoreType.DMA((4,)),
            pltpu.SemaphoreType.DMA((4,)),
        ],
        compiler_params=pltpu.CompilerParams(collective_id=0),
    )(x_bf, w_rel, ew_bf)


# device time: 87105 ns/iter; 1.9966x vs baseline; 1.0147x over previous
import jax
import jax.numpy as jnp
from jax import lax
from jax.experimental import pallas as pl
from jax.experimental.pallas import tpu as pltpu

N_DEV = 4
E_TOTAL = 16
E_LOC = 4


def kernel(x, router_W, route_idx, expert_W):
    n_tok, d_model = x.shape
    e_loc, _, d_hid = expert_W.shape

    scores = x @ router_W
    x_bf = x.astype(jnp.bfloat16)
    ew_bf = expert_W.astype(jnp.bfloat16)

    def body(x_ref, s_ref, idx_ref, ew_ref, out_ref, wall_ref, send_sems, recv_sems):
        my = lax.axis_index("i")
        left = lax.rem(my + N_DEV - 1, N_DEV)
        right = lax.rem(my + 1, N_DEV)

        barrier_sem = pltpu.get_barrier_semaphore()
        for nbr in (left, right):
            pl.semaphore_signal(
                barrier_sem, inc=1,
                device_id=(nbr,), device_id_type=pl.DeviceIdType.MESH,
            )
        pl.semaphore_wait(barrier_sem, 2)

        def rdma(src, dst, sem, dev):
            return pltpu.make_async_remote_copy(
                src_ref=src, dst_ref=dst,
                send_sem=send_sems.at[sem], recv_sem=recv_sems.at[sem],
                device_id=(dev,), device_id_type=pl.DeviceIdType.MESH,
            )

        h0Ra = rdma(ew_ref.at[pl.ds(0, 2)], wall_ref.at[pl.ds(8, 2)], 0, right)
        h0La = rdma(ew_ref.at[pl.ds(2, 2)], wall_ref.at[pl.ds(2, 2)], 1, left)
        h0Rb = rdma(ew_ref.at[pl.ds(2, 2)], wall_ref.at[pl.ds(10, 2)], 2, right)
        h0Lb = rdma(ew_ref.at[pl.ds(0, 2)], wall_ref.at[pl.ds(0, 2)], 3, left)
        h0Ra.start()
        h0La.start()
        h0Rb.start()
        h0Lb.start()

        s = s_ref[...]
        probs = jnp.exp(s - jnp.max(s, axis=-1, keepdims=True))
        probs = probs / jnp.sum(probs, axis=-1, keepdims=True)
        eids = lax.broadcasted_iota(jnp.int32, (n_tok, E_TOTAL), 1)
        i0 = idx_ref[:, 0:1]
        i1 = idx_ref[:, 1:2]
        g0 = jnp.sum(jnp.where(eids == i0, probs, 0.0), axis=-1, keepdims=True)
        g1 = jnp.sum(jnp.where(eids == i1, probs, 0.0), axis=-1, keepdims=True)
        r0 = lax.rem(i0 - E_LOC * my + E_TOTAL, E_TOTAL)
        r1 = lax.rem(i1 - E_LOC * my + E_TOTAL, E_TOTAL)
        inv = 1.0 / (g0 + g1)
        w = jnp.where(eids == r0, g0 * inv, 0.0) + jnp.where(
            eids == r1, g1 * inv, 0.0
        )

        xb = x_ref[...]
        acc = jnp.zeros((n_tok, d_hid), jnp.float32)

        def pair_mm(acc, ref, row, col):
            for k in range(2):
                y = jnp.dot(xb, ref[row + k], preferred_element_type=jnp.float32)
                acc = acc + w[:, col + k : col + k + 1] * y
            return acc

        acc = pair_mm(acc, ew_ref, 0, 0)
        acc = pair_mm(acc, ew_ref, 2, 2)

        h0Ra.wait_recv()
        d1R = rdma(wall_ref.at[pl.ds(8, 2)], wall_ref.at[pl.ds(4, 2)], 4, right)
        d1R.start()
        h0La.wait_recv()
        d1L = rdma(wall_ref.at[pl.ds(2, 2)], wall_ref.at[pl.ds(6, 2)], 5, left)
        d1L.start()

        acc = pair_mm(acc, wall_ref, 8, 12)
        acc = pair_mm(acc, wall_ref, 2, 6)

        h0Rb.wait_recv()
        acc = pair_mm(acc, wall_ref, 10, 14)
        h0Lb.wait_recv()
        acc = pair_mm(acc, wall_ref, 0, 4)

        d1R.wait_recv()
        acc = pair_mm(acc, wall_ref, 4, 8)
        d1L.wait_recv()
        acc = pair_mm(acc, wall_ref, 6, 10)

        out_ref[...] = acc

        for d in (h0Ra, h0La, h0Rb, h0Lb, d1R, d1L):
            d.wait_send()

    return pl.pallas_call(
        body,
        out_shape=jax.ShapeDtypeStruct((n_tok, d_hid), jnp.float32),
        in_specs=[
            pl.BlockSpec(memory_space=pltpu.VMEM),
            pl.BlockSpec(memory_space=pltpu.VMEM),
            pl.BlockSpec(memory_space=pltpu.VMEM),
            pl.BlockSpec(memory_space=pltpu.VMEM),
        ],
        out_specs=pl.BlockSpec(memory_space=pltpu.VMEM),
        scratch_shapes=[
            pltpu.VMEM((12, d_model, d_hid), jnp.bfloat16),
            pltpu.SemaphoreType.DMA((6,)),
            pltpu.SemaphoreType.DMA((6,)),
        ],
        compiler_params=pltpu.CompilerParams(collective_id=0),
    )(x_bf, scores, route_idx, ew_bf)


# device time: 77176 ns/iter; 2.2535x vs baseline; 1.1287x over previous
import jax
import jax.numpy as jnp
from jax import lax
from jax.experimental import pallas as pl
from jax.experimental.pallas import tpu as pltpu

N_DEV = 4
E_TOTAL = 16
E_LOC = 4


def kernel(x, router_W, route_idx, expert_W):
    n_tok, d_model = x.shape
    e_loc, _, d_hid = expert_W.shape
    h2 = d_hid // 2
    f2 = d_model // 2

    scores = x @ router_W
    probs = jax.nn.softmax(scores, axis=-1)
    oh0 = jax.nn.one_hot(route_idx[:, 0], E_TOTAL, dtype=jnp.float32)
    oh1 = jax.nn.one_hot(route_idx[:, 1], E_TOTAL, dtype=jnp.float32)
    g0 = (probs * oh0).sum(axis=-1, keepdims=True)
    g1 = (probs * oh1).sum(axis=-1, keepdims=True)
    gs = g0 + g1
    aux = jnp.concatenate(
        [
            route_idx[:, 0:1].astype(jnp.float32),
            route_idx[:, 1:2].astype(jnp.float32),
            g0 / gs,
            g1 / gs,
        ],
        axis=1,
    ).astype(jnp.bfloat16)

    x_bf = x.astype(jnp.bfloat16)
    ew_bf = expert_W.astype(jnp.bfloat16)

    def body(
        x_ref, aux_ref, ew_ref, out_ref,
        xw_ref,
        aw_ref,
        stR_ref,
        stL_ref,
        rbR_ref,
        rbL_ref,
        send_sems, recv_sems,
    ):
        my = lax.axis_index("i")
        left = lax.rem(my + N_DEV - 1, N_DEV)
        right = lax.rem(my + 1, N_DEV)

        barrier_sem = pltpu.get_barrier_semaphore()
        for nbr in (left, right):
            pl.semaphore_signal(
                barrier_sem, inc=1,
                device_id=(nbr,), device_id_type=pl.DeviceIdType.MESH,
            )
        pl.semaphore_wait(barrier_sem, 2)

        def rdma(src, dst, sem, dev):
            return pltpu.make_async_remote_copy(
                src_ref=src, dst_ref=dst,
                send_sem=send_sems.at[sem], recv_sem=recv_sems.at[sem],
                device_id=(dev,), device_id_type=pl.DeviceIdType.MESH,
            )

        xh0R = rdma(x_ref, xw_ref.at[2], 0, right)
        xh0L = rdma(x_ref, xw_ref.at[0], 1, left)
        ah0R = rdma(aux_ref, aw_ref.at[2], 2, right)
        ah0L = rdma(aux_ref, aw_ref.at[0], 3, left)
        xh0R.start()
        xh0L.start()
        ah0R.start()
        ah0L.start()

        def partial(slot, c0, cw):
            if slot < 0:
                xc = x_ref[...]
                ax = aux_ref
            else:
                xc = xw_ref[slot]
                ax = aw_ref.at[slot]
            i0 = ax[:, 0:1].astype(jnp.float32)
            i1 = ax[:, 1:2].astype(jnp.float32)
            a0 = ax[:, 2:3].astype(jnp.float32)
            a1 = ax[:, 3:4].astype(jnp.float32)
            acc = jnp.zeros((n_tok, cw), jnp.float32)
            for j in range(E_LOC):
                ej = (E_LOC * my + j).astype(jnp.float32)
                gate = jnp.where(i0 == ej, a0, 0.0) + jnp.where(i1 == ej, a1, 0.0)
                y = jnp.dot(
                    xc, ew_ref[j, :, c0 : c0 + cw],
                    preferred_element_type=jnp.float32,
                )
                acc = acc + gate * y
            return acc

        p_me_L = partial(-1, 0, h2)
        p_me_R = partial(-1, h2, h2)

        xh0R.wait_recv()
        ah0R.wait_recv()
        xh1R = rdma(
            xw_ref.at[2, :, pl.ds(0, f2)], xw_ref.at[1, :, pl.ds(0, f2)],
            4, right,
        )
        ah1R = rdma(aw_ref.at[2], aw_ref.at[1], 5, right)
        xh1R.start()
        ah1R.start()
        stR_ref[...] = partial(2, 0, h2).astype(jnp.bfloat16)
        rsR0 = rdma(stR_ref, rbR_ref.at[0], 6, right)
        rsR0.start()

        xh0L.wait_recv()
        ah0L.wait_recv()
        xh1L = rdma(
            xw_ref.at[0, :, pl.ds(f2, f2)], xw_ref.at[1, :, pl.ds(f2, f2)],
            7, left,
        )
        xh1L.start()
        stL_ref[...] = partial(0, h2, h2).astype(jnp.bfloat16)
        rsL0 = rdma(stL_ref, rbL_ref.at[0], 8, left)
        rsL0.start()

        xh1R.wait_recv()
        xh1L.wait_recv()
        ah1R.wait_recv()
        p_d_L = partial(1, 0, h2).astype(jnp.bfloat16)
        p_d_R = partial(1, h2, h2).astype(jnp.bfloat16)

        rsR0.wait_recv()
        rbR_ref[0] = rbR_ref[0] + p_d_L
        rsR1 = rdma(rbR_ref.at[0], rbR_ref.at[1], 9, right)
        rsR1.start()
        rsL0.wait_recv()
        rbL_ref[0] = rbL_ref[0] + p_d_R
        rsL1 = rdma(rbL_ref.at[0], rbL_ref.at[1], 10, left)
        rsL1.start()

        p_n_L = partial(0, 0, h2).astype(jnp.bfloat16)
        p_n_R = partial(2, h2, h2).astype(jnp.bfloat16)

        rsR1.wait_recv()
        rbR_ref[1] = rbR_ref[1] + p_n_L
        rsR2 = rdma(rbR_ref.at[1], rbR_ref.at[2], 11, right)
        rsR2.start()
        rsL1.wait_recv()
        rbL_ref[1] = rbL_ref[1] + p_n_R
        rsL2 = rdma(rbL_ref.at[1], rbL_ref.at[2], 12, left)
        rsL2.start()

        rsR2.wait_recv()
        out_ref[:, 0:h2] = rbR_ref[2].astype(jnp.float32) + p_me_L
        rsL2.wait_recv()
        out_ref[:, h2:d_hid] = rbL_ref[2].astype(jnp.float32) + p_me_R

        for d in (
            xh0R, xh0L, ah0R, ah0L, xh1R, ah1R, rsR0, xh1L, rsL0,
            rsR1, rsL1, rsR2, rsL2,
        ):
            d.wait_send()

    return pl.pallas_call(
        body,
        out_shape=jax.ShapeDtypeStruct((n_tok, d_hid), jnp.float32),
        in_specs=[
            pl.BlockSpec(memory_space=pltpu.VMEM),
            pl.BlockSpec(memory_space=pltpu.VMEM),
            pl.BlockSpec(memory_space=pltpu.VMEM),
        ],
        out_specs=pl.BlockSpec(memory_space=pltpu.VMEM),
        scratch_shapes=[
            pltpu.VMEM((3, n_tok, d_model), jnp.bfloat16),
            pltpu.VMEM((3, n_tok, 4), jnp.bfloat16),
            pltpu.VMEM((n_tok, h2), jnp.bfloat16),
            pltpu.VMEM((n_tok, h2), jnp.bfloat16),
            pltpu.VMEM((3, n_tok, h2), jnp.bfloat16),
            pltpu.VMEM((3, n_tok, h2), jnp.bfloat16),
            pltpu.SemaphoreType.DMA((13,)),
            pltpu.SemaphoreType.DMA((13,)),
        ],
        compiler_params=pltpu.CompilerParams(collective_id=0),
    )(x_bf, aux, ew_bf)


# device time: 74955 ns/iter; 2.3203x vs baseline; 1.0296x over previous
import jax
import jax.numpy as jnp
from jax import lax
from jax.experimental import pallas as pl
from jax.experimental.pallas import tpu as pltpu

N_DEV = 4
E_TOTAL = 16
E_LOC = 4


def kernel(x, router_W, route_idx, expert_W):
    n_tok, d_model = x.shape
    e_loc, _, d_hid = expert_W.shape
    h2 = d_hid // 2
    f2 = d_model // 2

    def body(
        x_ref, rw_ref, idx_ref, ew_ref, out_ref,
        xb_ref,
        auxs_ref,
        ewb_ref,
        xw_ref,
        aw_ref,
        stR_ref,
        stL_ref,
        rbR_ref,
        rbL_ref,
        send_sems, recv_sems,
    ):
        my = lax.axis_index("i")
        left = lax.rem(my + N_DEV - 1, N_DEV)
        right = lax.rem(my + 1, N_DEV)

        barrier_sem = pltpu.get_barrier_semaphore()
        for nbr in (left, right):
            pl.semaphore_signal(
                barrier_sem, inc=1,
                device_id=(nbr,), device_id_type=pl.DeviceIdType.MESH,
            )
        pl.semaphore_wait(barrier_sem, 2)

        def rdma(src, dst, sem, dev):
            return pltpu.make_async_remote_copy(
                src_ref=src, dst_ref=dst,
                send_sem=send_sems.at[sem], recv_sem=recv_sems.at[sem],
                device_id=(dev,), device_id_type=pl.DeviceIdType.MESH,
            )

        xb = x_ref[...].astype(jnp.bfloat16)
        xb_ref[...] = xb
        xh0R = rdma(xb_ref, xw_ref.at[2], 0, right)
        xh0L = rdma(xb_ref, xw_ref.at[0], 1, left)
        xh0R.start()
        xh0L.start()

        scores = jnp.dot(
            xb, rw_ref[...].astype(jnp.bfloat16),
            preferred_element_type=jnp.float32,
        )
        p = jnp.exp(scores - jnp.max(scores, axis=-1, keepdims=True))
        p = p / jnp.sum(p, axis=-1, keepdims=True)
        eids = lax.broadcasted_iota(jnp.int32, (n_tok, E_TOTAL), 1)
        i0 = idx_ref[:, 0:1]
        i1 = idx_ref[:, 1:2]
        g0 = jnp.sum(jnp.where(eids == i0, p, 0.0), axis=-1, keepdims=True)
        g1 = jnp.sum(jnp.where(eids == i1, p, 0.0), axis=-1, keepdims=True)
        inv = 1.0 / (g0 + g1)
        auxs_ref[...] = jnp.concatenate(
            [i0.astype(jnp.float32), i1.astype(jnp.float32), g0 * inv, g1 * inv],
            axis=1,
        ).astype(jnp.bfloat16)
        ah0R = rdma(auxs_ref, aw_ref.at[2], 2, right)
        ah0L = rdma(auxs_ref, aw_ref.at[0], 3, left)
        ah0R.start()
        ah0L.start()

        ewb_ref[...] = ew_ref[...].astype(jnp.bfloat16)

        def partial(slot, c0, cw):
            if slot < 0:
                xc = xb
                ax = auxs_ref
            else:
                xc = xw_ref[slot]
                ax = aw_ref.at[slot]
            a_i0 = ax[:, 0:1].astype(jnp.float32)
            a_i1 = ax[:, 1:2].astype(jnp.float32)
            a_g0 = ax[:, 2:3].astype(jnp.float32)
            a_g1 = ax[:, 3:4].astype(jnp.float32)
            acc = jnp.zeros((n_tok, cw), jnp.float32)
            for j in range(E_LOC):
                ej = (E_LOC * my + j).astype(jnp.float32)
                gate = jnp.where(a_i0 == ej, a_g0, 0.0) + jnp.where(
                    a_i1 == ej, a_g1, 0.0
                )
                y = jnp.dot(
                    xc, ewb_ref[j, :, c0 : c0 + cw],
                    preferred_element_type=jnp.float32,
                )
                acc = acc + gate * y
            return acc

        p_me_L = partial(-1, 0, h2)
        p_me_R = partial(-1, h2, h2)

        xh0R.wait_recv()
        ah0R.wait_recv()
        xh1R = rdma(
            xw_ref.at[2, :, pl.ds(0, f2)], xw_ref.at[1, :, pl.ds(0, f2)],
            4, right,
        )
        ah1R = rdma(aw_ref.at[2], aw_ref.at[1], 5, right)
        xh1R.start()
        ah1R.start()
        stR_ref[...] = partial(2, 0, h2).astype(jnp.bfloat16)
        rsR0 = rdma(stR_ref, rbR_ref.at[0], 6, right)
        rsR0.start()

        xh0L.wait_recv()
        ah0L.wait_recv()
        xh1L = rdma(
            xw_ref.at[0, :, pl.ds(f2, f2)], xw_ref.at[1, :, pl.ds(f2, f2)],
            7, left,
        )
        xh1L.start()
        stL_ref[...] = partial(0, h2, h2).astype(jnp.bfloat16)
        rsL0 = rdma(stL_ref, rbL_ref.at[0], 8, left)
        rsL0.start()

        xh1R.wait_recv()
        xh1L.wait_recv()
        ah1R.wait_recv()
        p_d_L = partial(1, 0, h2).astype(jnp.bfloat16)
        p_d_R = partial(1, h2, h2).astype(jnp.bfloat16)

        rsR0.wait_recv()
        rbR_ref[0] = rbR_ref[0] + p_d_L
        rsR1 = rdma(rbR_ref.at[0], rbR_ref.at[1], 9, right)
        rsR1.start()
        rsL0.wait_recv()
        rbL_ref[0] = rbL_ref[0] + p_d_R
        rsL1 = rdma(rbL_ref.at[0], rbL_ref.at[1], 10, left)
        rsL1.start()

        p_n_L = partial(0, 0, h2).astype(jnp.bfloat16)
        p_n_R = partial(2, h2, h2).astype(jnp.bfloat16)

        rsR1.wait_recv()
        rbR_ref[1] = rbR_ref[1] + p_n_L
        rsR2 = rdma(rbR_ref.at[1], rbR_ref.at[2], 11, right)
        rsR2.start()
        rsL1.wait_recv()
        rbL_ref[1] = rbL_ref[1] + p_n_R
        rsL2 = rdma(rbL_ref.at[1], rbL_ref.at[2], 12, left)
        rsL2.start()

        rsR2.wait_recv()
        out_ref[:, 0:h2] = (
            rbR_ref[2].astype(jnp.float32) + p_me_L
        ).astype(jnp.bfloat16)
        rsL2.wait_recv()
        out_ref[:, h2:d_hid] = (
            rbL_ref[2].astype(jnp.float32) + p_me_R
        ).astype(jnp.bfloat16)

        for d in (
            xh0R, xh0L, ah0R, ah0L, xh1R, ah1R, rsR0, xh1L, rsL0,
            rsR1, rsL1, rsR2, rsL2,
        ):
            d.wait_send()

    return pl.pallas_call(
        body,
        out_shape=jax.ShapeDtypeStruct((n_tok, d_hid), jnp.bfloat16),
        in_specs=[
            pl.BlockSpec(memory_space=pltpu.VMEM),
            pl.BlockSpec(memory_space=pltpu.VMEM),
            pl.BlockSpec(memory_space=pltpu.VMEM),
            pl.BlockSpec(memory_space=pltpu.VMEM),
        ],
        out_specs=pl.BlockSpec(memory_space=pltpu.VMEM),
        scratch_shapes=[
            pltpu.VMEM((n_tok, d_model), jnp.bfloat16),
            pltpu.VMEM((n_tok, 4), jnp.bfloat16),
            pltpu.VMEM((e_loc, d_model, d_hid), jnp.bfloat16),
            pltpu.VMEM((3, n_tok, d_model), jnp.bfloat16),
            pltpu.VMEM((3, n_tok, 4), jnp.bfloat16),
            pltpu.VMEM((n_tok, h2), jnp.bfloat16),
            pltpu.VMEM((n_tok, h2), jnp.bfloat16),
            pltpu.VMEM((3, n_tok, h2), jnp.bfloat16),
            pltpu.VMEM((3, n_tok, h2), jnp.bfloat16),
            pltpu.SemaphoreType.DMA((13,)),
            pltpu.SemaphoreType.DMA((13,)),
        ],
        compiler_params=pltpu.CompilerParams(collective_id=0),
    )(x, router_W, route_idx, expert_W)


# device time: 72749 ns/iter; 2.3906x vs baseline; 1.0303x over previous
import jax
import jax.numpy as jnp
from jax import lax
from jax.experimental import pallas as pl
from jax.experimental.pallas import tpu as pltpu

N_DEV = 4
E_TOTAL = 16
E_LOC = 4


def kernel(x, router_W, route_idx, expert_W):
    n_tok, d_model = x.shape
    e_loc, _, d_hid = expert_W.shape
    h2 = d_hid // 2
    q = d_hid // 4
    f2 = d_model // 2

    def body(
        x_ref, rw_ref, idx_ref, ew_ref, out_ref,
        xb_ref,
        auxs_ref,
        ewf_ref,
        ewb_ref,
        xw_ref,
        aw_ref,
        stR_ref,
        stL_ref,
        rbR_ref,
        rbL_ref,
        ew_sem, send_sems, recv_sems,
    ):
        my = lax.axis_index("i")
        left = lax.rem(my + N_DEV - 1, N_DEV)
        right = lax.rem(my + 1, N_DEV)

        ew_cp = pltpu.make_async_copy(ew_ref, ewf_ref, ew_sem)
        ew_cp.start()

        barrier_sem = pltpu.get_barrier_semaphore()
        for nbr in (left, right):
            pl.semaphore_signal(
                barrier_sem, inc=1,
                device_id=(nbr,), device_id_type=pl.DeviceIdType.MESH,
            )
        pl.semaphore_wait(barrier_sem, 2)

        def rdma(src, dst, sem, dev):
            return pltpu.make_async_remote_copy(
                src_ref=src, dst_ref=dst,
                send_sem=send_sems.at[sem], recv_sem=recv_sems.at[sem],
                device_id=(dev,), device_id_type=pl.DeviceIdType.MESH,
            )

        xb = x_ref[...].astype(jnp.bfloat16)
        xb_ref[...] = xb
        xh0R = rdma(xb_ref, xw_ref.at[2], 0, right)
        xh0L = rdma(xb_ref, xw_ref.at[0], 1, left)
        xh0R.start()
        xh0L.start()

        scores = jnp.dot(
            xb, rw_ref[...].astype(jnp.bfloat16),
            preferred_element_type=jnp.float32,
        )
        p = jnp.exp(scores - jnp.max(scores, axis=-1, keepdims=True))
        p = p / jnp.sum(p, axis=-1, keepdims=True)
        eids = lax.broadcasted_iota(jnp.int32, (n_tok, E_TOTAL), 1)
        i0 = idx_ref[:, 0:1]
        i1 = idx_ref[:, 1:2]
        g0 = jnp.sum(jnp.where(eids == i0, p, 0.0), axis=-1, keepdims=True)
        g1 = jnp.sum(jnp.where(eids == i1, p, 0.0), axis=-1, keepdims=True)
        inv = 1.0 / (g0 + g1)
        auxs_ref[...] = jnp.concatenate(
            [i0.astype(jnp.float32), i1.astype(jnp.float32), g0 * inv, g1 * inv],
            axis=1,
        ).astype(jnp.bfloat16)
        ah0R = rdma(auxs_ref, aw_ref.at[2], 2, right)
        ah0L = rdma(auxs_ref, aw_ref.at[0], 3, left)
        ah0R.start()
        ah0L.start()

        ew_cp.wait()
        ewb_ref[...] = ewf_ref[...].astype(jnp.bfloat16)

        def partial(slot, c0, cw):
            if slot < 0:
                xc = xb
                ax = auxs_ref
            else:
                xc = xw_ref[slot]
                ax = aw_ref.at[slot]
            a_i0 = ax[:, 0:1].astype(jnp.float32)
            a_i1 = ax[:, 1:2].astype(jnp.float32)
            a_g0 = ax[:, 2:3].astype(jnp.float32)
            a_g1 = ax[:, 3:4].astype(jnp.float32)
            acc = jnp.zeros((n_tok, cw), jnp.float32)
            for j in range(E_LOC):
                ej = (E_LOC * my + j).astype(jnp.float32)
                gate = jnp.where(a_i0 == ej, a_g0, 0.0) + jnp.where(
                    a_i1 == ej, a_g1, 0.0
                )
                y = jnp.dot(
                    xc, ewb_ref[j, :, c0 : c0 + cw],
                    preferred_element_type=jnp.float32,
                )
                acc = acc + gate * y
            return acc

        p_me_L = partial(-1, 0, h2)
        p_me_R = partial(-1, h2, h2)

        xh0R.wait_recv()
        ah0R.wait_recv()
        xh1R = rdma(
            xw_ref.at[2, :, pl.ds(0, f2)], xw_ref.at[1, :, pl.ds(0, f2)],
            4, right,
        )
        ah1R = rdma(aw_ref.at[2], aw_ref.at[1], 5, right)
        xh1R.start()
        ah1R.start()
        xh0L.wait_recv()
        ah0L.wait_recv()
        xh1L = rdma(
            xw_ref.at[0, :, pl.ds(f2, f2)], xw_ref.at[1, :, pl.ds(f2, f2)],
            6, left,
        )
        xh1L.start()

        stR_ref[:, 0:q] = partial(2, 0, q).astype(jnp.bfloat16)
        rsR0a = rdma(stR_ref.at[:, pl.ds(0, q)], rbR_ref.at[0, :, pl.ds(0, q)], 7, right)
        rsR0a.start()
        stL_ref[:, 0:q] = partial(0, h2, q).astype(jnp.bfloat16)
        rsL0a = rdma(stL_ref.at[:, pl.ds(0, q)], rbL_ref.at[0, :, pl.ds(0, q)], 8, left)
        rsL0a.start()
        stR_ref[:, q:h2] = partial(2, q, q).astype(jnp.bfloat16)
        rsR0b = rdma(stR_ref.at[:, pl.ds(q, q)], rbR_ref.at[0, :, pl.ds(q, q)], 9, right)
        rsR0b.start()
        stL_ref[:, q:h2] = partial(0, h2 + q, q).astype(jnp.bfloat16)
        rsL0b = rdma(stL_ref.at[:, pl.ds(q, q)], rbL_ref.at[0, :, pl.ds(q, q)], 10, left)
        rsL0b.start()

        xh1R.wait_recv()
        xh1L.wait_recv()
        ah1R.wait_recv()

        p_dL0 = partial(1, 0, q).astype(jnp.bfloat16)
        rsR0a.wait_recv()
        rbR_ref[0, :, 0:q] = rbR_ref[0, :, 0:q] + p_dL0
        rsR1a = rdma(rbR_ref.at[0, :, pl.ds(0, q)], rbR_ref.at[1, :, pl.ds(0, q)], 11, right)
        rsR1a.start()

        p_dR0 = partial(1, h2, q).astype(jnp.bfloat16)
        rsL0a.wait_recv()
        rbL_ref[0, :, 0:q] = rbL_ref[0, :, 0:q] + p_dR0
        rsL1a = rdma(rbL_ref.at[0, :, pl.ds(0, q)], rbL_ref.at[1, :, pl.ds(0, q)], 12, left)
        rsL1a.start()

        p_dL1 = partial(1, q, q).astype(jnp.bfloat16)
        rsR0b.wait_recv()
        rbR_ref[0, :, q:h2] = rbR_ref[0, :, q:h2] + p_dL1
        rsR1b = rdma(rbR_ref.at[0, :, pl.ds(q, q)], rbR_ref.at[1, :, pl.ds(q, q)], 13, right)
        rsR1b.start()

        p_dR1 = partial(1, h2 + q, q).astype(jnp.bfloat16)
        rsL0b.wait_recv()
        rbL_ref[0, :, q:h2] = rbL_ref[0, :, q:h2] + p_dR1
        rsL1b = rdma(rbL_ref.at[0, :, pl.ds(q, q)], rbL_ref.at[1, :, pl.ds(q, q)], 14, left)
        rsL1b.start()

        p_nL0 = partial(0, 0, q).astype(jnp.bfloat16)
        rsR1a.wait_recv()
        rbR_ref[1, :, 0:q] = rbR_ref[1, :, 0:q] + p_nL0
        rsR2a = rdma(rbR_ref.at[1, :, pl.ds(0, q)], rbR_ref.at[2, :, pl.ds(0, q)], 15, right)
        rsR2a.start()

        p_nR0 = partial(2, h2, q).astype(jnp.bfloat16)
        rsL1a.wait_recv()
        rbL_ref[1, :, 0:q] = rbL_ref[1, :, 0:q] + p_nR0
        rsL2a = rdma(rbL_ref.at[1, :, pl.ds(0, q)], rbL_ref.at[2, :, pl.ds(0, q)], 16, left)
        rsL2a.start()

        p_nL1 = partial(0, q, q).astype(jnp.bfloat16)
        rsR1b.wait_recv()
        rbR_ref[1, :, q:h2] = rbR_ref[1, :, q:h2] + p_nL1
        rsR2b = rdma(rbR_ref.at[1, :, pl.ds(q, q)], rbR_ref.at[2, :, pl.ds(q, q)], 17, right)
        rsR2b.start()

        p_nR1 = partial(2, h2 + q, q).astype(jnp.bfloat16)
        rsL1b.wait_recv()
        rbL_ref[1, :, q:h2] = rbL_ref[1, :, q:h2] + p_nR1
        rsL2b = rdma(rbL_ref.at[1, :, pl.ds(q, q)], rbL_ref.at[2, :, pl.ds(q, q)], 18, left)
        rsL2b.start()

        rsR2a.wait_recv()
        out_ref[:, 0:q] = (
            rbR_ref[2, :, 0:q].astype(jnp.float32) + p_me_L[:, 0:q]
        ).astype(jnp.bfloat16)
        rsL2a.wait_recv()
        out_ref[:, h2 : h2 + q] = (
            rbL_ref[2, :, 0:q].astype(jnp.float32) + p_me_R[:, 0:q]
        ).astype(jnp.bfloat16)
        rsR2b.wait_recv()
        out_ref[:, q:h2] = (
            rbR_ref[2, :, q:h2].astype(jnp.float32) + p_me_L[:, q:h2]
        ).astype(jnp.bfloat16)
        rsL2b.wait_recv()
        out_ref[:, h2 + q : d_hid] = (
            rbL_ref[2, :, q:h2].astype(jnp.float32) + p_me_R[:, q:h2]
        ).astype(jnp.bfloat16)

        for d in (
            xh0R, xh0L, ah0R, ah0L, xh1R, ah1R, xh1L,
            rsR0a, rsL0a, rsR0b, rsL0b,
            rsR1a, rsL1a, rsR1b, rsL1b,
            rsR2a, rsL2a, rsR2b, rsL2b,
        ):
            d.wait_send()

    return pl.pallas_call(
        body,
        out_shape=jax.ShapeDtypeStruct((n_tok, d_hid), jnp.bfloat16),
        in_specs=[
            pl.BlockSpec(memory_space=pltpu.VMEM),
            pl.BlockSpec(memory_space=pltpu.VMEM),
            pl.BlockSpec(memory_space=pltpu.VMEM),
            pl.BlockSpec(memory_space=pl.ANY),
        ],
        out_specs=pl.BlockSpec(memory_space=pltpu.VMEM),
        scratch_shapes=[
            pltpu.VMEM((n_tok, d_model), jnp.bfloat16),
            pltpu.VMEM((n_tok, 4), jnp.bfloat16),
            pltpu.VMEM((e_loc, d_model, d_hid), jnp.float32),
            pltpu.VMEM((e_loc, d_model, d_hid), jnp.bfloat16),
            pltpu.VMEM((3, n_tok, d_model), jnp.bfloat16),
            pltpu.VMEM((3, n_tok, 4), jnp.bfloat16),
            pltpu.VMEM((n_tok, h2), jnp.bfloat16),
            pltpu.VMEM((n_tok, h2), jnp.bfloat16),
            pltpu.VMEM((3, n_tok, h2), jnp.bfloat16),
            pltpu.VMEM((3, n_tok, h2), jnp.bfloat16),
            pltpu.SemaphoreType.DMA,
            pltpu.SemaphoreType.DMA((19,)),
            pltpu.SemaphoreType.DMA((19,)),
        ],
        compiler_params=pltpu.CompilerParams(
            collective_id=0, vmem_limit_bytes=64 * 1024 * 1024
        ),
    )(x, router_W, route_idx, expert_W)


# device time: 72413 ns/iter; 2.4017x vs baseline; 1.0046x over previous
import jax
import jax.numpy as jnp
from jax import lax
from jax.experimental import pallas as pl
from jax.experimental.pallas import tpu as pltpu

N_DEV = 4
E_TOTAL = 16
E_LOC = 4


def kernel(x, router_W, route_idx, expert_W):
    n_tok, d_model = x.shape
    e_loc, _, d_hid = expert_W.shape
    h2 = d_hid // 2
    q = d_hid // 4
    f2 = d_model // 2

    scores = x @ router_W
    probs = jax.nn.softmax(scores, axis=-1)
    oh0 = jax.nn.one_hot(route_idx[:, 0], E_TOTAL, dtype=jnp.float32)
    oh1 = jax.nn.one_hot(route_idx[:, 1], E_TOTAL, dtype=jnp.float32)
    g0 = (probs * oh0).sum(axis=-1, keepdims=True)
    g1 = (probs * oh1).sum(axis=-1, keepdims=True)
    inv = 1.0 / (g0 + g1)
    aux = jnp.concatenate(
        [
            route_idx[:, 0:1].astype(jnp.float32),
            route_idx[:, 1:2].astype(jnp.float32),
            g0 * inv,
            g1 * inv,
        ],
        axis=1,
    ).astype(jnp.bfloat16)

    def body(
        x_ref, aux_ref, ew_ref, out_ref,
        xb_ref,
        ewf_ref,
        ewb_ref,
        xw_ref,
        aw_ref,
        stR_ref,
        stL_ref,
        rbR_ref,
        rbL_ref,
        outv_ref,
        ew_sem, out_sems, send_sems, recv_sems,
    ):
        my = lax.axis_index("i")
        left = lax.rem(my + N_DEV - 1, N_DEV)
        right = lax.rem(my + 1, N_DEV)

        ew_cp = pltpu.make_async_copy(ew_ref, ewf_ref, ew_sem)
        ew_cp.start()

        barrier_sem = pltpu.get_barrier_semaphore()
        for nbr in (left, right):
            pl.semaphore_signal(
                barrier_sem, inc=1,
                device_id=(nbr,), device_id_type=pl.DeviceIdType.MESH,
            )
        pl.semaphore_wait(barrier_sem, 2)

        def rdma(src, dst, sem, dev):
            return pltpu.make_async_remote_copy(
                src_ref=src, dst_ref=dst,
                send_sem=send_sems.at[sem], recv_sem=recv_sems.at[sem],
                device_id=(dev,), device_id_type=pl.DeviceIdType.MESH,
            )

        xb = x_ref[...].astype(jnp.bfloat16)
        xb_ref[...] = xb
        xh0R = rdma(xb_ref, xw_ref.at[2], 0, right)
        xh0L = rdma(xb_ref, xw_ref.at[0], 1, left)
        ah0R = rdma(aux_ref, aw_ref.at[2], 2, right)
        ah0L = rdma(aux_ref, aw_ref.at[0], 3, left)
        xh0R.start()
        xh0L.start()
        ah0R.start()
        ah0L.start()

        ew_cp.wait()
        ewb_ref[...] = ewf_ref[...].astype(jnp.bfloat16)

        def partial(slot, c0, cw):
            if slot < 0:
                xc = xb
                ax = aux_ref
            else:
                xc = xw_ref[slot]
                ax = aw_ref.at[slot]
            a_i0 = ax[:, 0:1].astype(jnp.float32)
            a_i1 = ax[:, 1:2].astype(jnp.float32)
            a_g0 = ax[:, 2:3].astype(jnp.float32)
            a_g1 = ax[:, 3:4].astype(jnp.float32)
            acc = jnp.zeros((n_tok, cw), jnp.float32)
            for j in range(E_LOC):
                ej = (E_LOC * my + j).astype(jnp.float32)
                gate = jnp.where(a_i0 == ej, a_g0, 0.0) + jnp.where(
                    a_i1 == ej, a_g1, 0.0
                )
                y = jnp.dot(
                    xc, ewb_ref[j, :, c0 : c0 + cw],
                    preferred_element_type=jnp.float32,
                )
                acc = acc + gate * y
            return acc

        p_me_L = partial(-1, 0, h2)
        p_me_R = partial(-1, h2, h2)

        xh0R.wait_recv()
        ah0R.wait_recv()
        xh1R = rdma(
            xw_ref.at[2, :, pl.ds(0, f2)], xw_ref.at[1, :, pl.ds(0, f2)],
            4, right,
        )
        ah1R = rdma(aw_ref.at[2], aw_ref.at[1], 5, right)
        xh1R.start()
        ah1R.start()
        xh0L.wait_recv()
        ah0L.wait_recv()
        xh1L = rdma(
            xw_ref.at[0, :, pl.ds(f2, f2)], xw_ref.at[1, :, pl.ds(f2, f2)],
            6, left,
        )
        xh1L.start()

        stR_ref[:, 0:q] = partial(2, 0, q).astype(jnp.bfloat16)
        rsR0a = rdma(stR_ref.at[:, pl.ds(0, q)], rbR_ref.at[0, :, pl.ds(0, q)], 7, right)
        rsR0a.start()
        stL_ref[:, 0:q] = partial(0, h2, q).astype(jnp.bfloat16)
        rsL0a = rdma(stL_ref.at[:, pl.ds(0, q)], rbL_ref.at[0, :, pl.ds(0, q)], 8, left)
        rsL0a.start()
        stR_ref[:, q:h2] = partial(2, q, q).astype(jnp.bfloat16)
        rsR0b = rdma(stR_ref.at[:, pl.ds(q, q)], rbR_ref.at[0, :, pl.ds(q, q)], 9, right)
        rsR0b.start()
        stL_ref[:, q:h2] = partial(0, h2 + q, q).astype(jnp.bfloat16)
        rsL0b = rdma(stL_ref.at[:, pl.ds(q, q)], rbL_ref.at[0, :, pl.ds(q, q)], 10, left)
        rsL0b.start()

        xh1R.wait_recv()
        xh1L.wait_recv()
        ah1R.wait_recv()

        p_dL0 = partial(1, 0, q).astype(jnp.bfloat16)
        rsR0a.wait_recv()
        rbR_ref[0, :, 0:q] = rbR_ref[0, :, 0:q] + p_dL0
        rsR1a = rdma(rbR_ref.at[0, :, pl.ds(0, q)], rbR_ref.at[1, :, pl.ds(0, q)], 11, right)
        rsR1a.start()

        p_dR0 = partial(1, h2, q).astype(jnp.bfloat16)
        rsL0a.wait_recv()
        rbL_ref[0, :, 0:q] = rbL_ref[0, :, 0:q] + p_dR0
        rsL1a = rdma(rbL_ref.at[0, :, pl.ds(0, q)], rbL_ref.at[1, :, pl.ds(0, q)], 12, left)
        rsL1a.start()

        p_dL1 = partial(1, q, q).astype(jnp.bfloat16)
        rsR0b.wait_recv()
        rbR_ref[0, :, q:h2] = rbR_ref[0, :, q:h2] + p_dL1
        rsR1b = rdma(rbR_ref.at[0, :, pl.ds(q, q)], rbR_ref.at[1, :, pl.ds(q, q)], 13, right)
        rsR1b.start()

        p_dR1 = partial(1, h2 + q, q).astype(jnp.bfloat16)
        rsL0b.wait_recv()
        rbL_ref[0, :, q:h2] = rbL_ref[0, :, q:h2] + p_dR1
        rsL1b = rdma(rbL_ref.at[0, :, pl.ds(q, q)], rbL_ref.at[1, :, pl.ds(q, q)], 14, left)
        rsL1b.start()

        p_nL0 = partial(0, 0, q).astype(jnp.bfloat16)
        rsR1a.wait_recv()
        rbR_ref[1, :, 0:q] = rbR_ref[1, :, 0:q] + p_nL0
        rsR2a = rdma(rbR_ref.at[1, :, pl.ds(0, q)], rbR_ref.at[2, :, pl.ds(0, q)], 15, right)
        rsR2a.start()

        p_nR0 = partial(2, h2, q).astype(jnp.bfloat16)
        rsL1a.wait_recv()
        rbL_ref[1, :, 0:q] = rbL_ref[1, :, 0:q] + p_nR0
        rsL2a = rdma(rbL_ref.at[1, :, pl.ds(0, q)], rbL_ref.at[2, :, pl.ds(0, q)], 16, left)
        rsL2a.start()

        p_nL1 = partial(0, q, q).astype(jnp.bfloat16)
        rsR1b.wait_recv()
        rbR_ref[1, :, q:h2] = rbR_ref[1, :, q:h2] + p_nL1
        rsR2b = rdma(rbR_ref.at[1, :, pl.ds(q, q)], rbR_ref.at[2, :, pl.ds(q, q)], 17, right)
        rsR2b.start()

        p_nR1 = partial(2, h2 + q, q).astype(jnp.bfloat16)
        rsL1b.wait_recv()
        rbL_ref[1, :, q:h2] = rbL_ref[1, :, q:h2] + p_nR1
        rsL2b = rdma(rbL_ref.at[1, :, pl.ds(q, q)], rbL_ref.at[2, :, pl.ds(q, q)], 18, left)
        rsL2b.start()

        def finish(idx, rb_ref, rbq0, outq0, p_me):
            outv_ref[:, outq0 : outq0 + q] = (
                rb_ref[2, :, rbq0 : rbq0 + q].astype(jnp.float32)
                + p_me[:, rbq0 : rbq0 + q]
            ).astype(jnp.bfloat16)
            cp = pltpu.make_async_copy(
                outv_ref.at[:, pl.ds(outq0, q)],
                out_ref.at[:, pl.ds(outq0, q)],
                out_sems.at[idx],
            )
            cp.start()
            return cp

        rsR2a.wait_recv()
        c0 = finish(0, rbR_ref, 0, 0, p_me_L)
        rsL2a.wait_recv()
        c1 = finish(1, rbL_ref, 0, h2, p_me_R)
        rsR2b.wait_recv()
        c2 = finish(2, rbR_ref, q, q, p_me_L)
        rsL2b.wait_recv()
        c3 = finish(3, rbL_ref, q, h2 + q, p_me_R)
        c0.wait()
        c1.wait()
        c2.wait()
        c3.wait()

        for d in (
            xh0R, xh0L, ah0R, ah0L, xh1R, ah1R, xh1L,
            rsR0a, rsL0a, rsR0b, rsL0b,
            rsR1a, rsL1a, rsR1b, rsL1b,
            rsR2a, rsL2a, rsR2b, rsL2b,
        ):
            d.wait_send()

    return pl.pallas_call(
        body,
        out_shape=jax.ShapeDtypeStruct((n_tok, d_hid), jnp.bfloat16),
        in_specs=[
            pl.BlockSpec(memory_space=pltpu.VMEM),
            pl.BlockSpec(memory_space=pltpu.VMEM),
            pl.BlockSpec(memory_space=pl.ANY),
        ],
        out_specs=pl.BlockSpec(memory_space=pl.ANY),
        scratch_shapes=[
            pltpu.VMEM((n_tok, d_model), jnp.bfloat16),
            pltpu.VMEM((e_loc, d_model, d_hid), jnp.float32),
            pltpu.VMEM((e_loc, d_model, d_hid), jnp.bfloat16),
            pltpu.VMEM((3, n_tok, d_model), jnp.bfloat16),
            pltpu.VMEM((3, n_tok, 4), jnp.bfloat16),
            pltpu.VMEM((n_tok, h2), jnp.bfloat16),
            pltpu.VMEM((n_tok, h2), jnp.bfloat16),
            pltpu.VMEM((3, n_tok, h2), jnp.bfloat16),
            pltpu.VMEM((3, n_tok, h2), jnp.bfloat16),
            pltpu.VMEM((n_tok, d_hid), jnp.bfloat16),
            pltpu.SemaphoreType.DMA,
            pltpu.SemaphoreType.DMA((4,)),
            pltpu.SemaphoreType.DMA((19,)),
            pltpu.SemaphoreType.DMA((19,)),
        ],
        compiler_params=pltpu.CompilerParams(
            collective_id=0, vmem_limit_bytes=64 * 1024 * 1024
        ),
    )(x, aux, expert_W)


# device time: 72243 ns/iter; 2.4074x vs baseline; 1.0024x over previous
import jax
import jax.numpy as jnp
from jax import lax
from jax.experimental import pallas as pl
from jax.experimental.pallas import tpu as pltpu

N_DEV = 4
E_TOTAL = 16
E_LOC = 4


def kernel(x, router_W, route_idx, expert_W):
    n_tok, d_model = x.shape
    e_loc, _, d_hid = expert_W.shape
    h2 = d_hid // 2
    q = d_hid // 4
    f2 = d_model // 2

    ridx_f = route_idx.astype(jnp.float32)
    rw_bf = router_W.astype(jnp.bfloat16)

    def body(
        x_ref, rw_ref, ridx_ref, ew_ref, out_ref,
        xb_ref,
        auxs_ref,
        ewf_ref,
        ewb_ref,
        xw_ref,
        aw_ref,
        stR_ref,
        stL_ref,
        rbR_ref,
        rbL_ref,
        outv_ref,
        ew_sem, out_sems, send_sems, recv_sems,
    ):
        my = lax.axis_index("i")
        left = lax.rem(my + N_DEV - 1, N_DEV)
        right = lax.rem(my + 1, N_DEV)

        ew_cp0 = pltpu.make_async_copy(
            ew_ref.at[pl.ds(0, 2)], ewf_ref.at[pl.ds(0, 2)], ew_sem
        )
        ew_cp0.start()

        barrier_sem = pltpu.get_barrier_semaphore()
        for nbr in (left, right):
            pl.semaphore_signal(
                barrier_sem, inc=1,
                device_id=(nbr,), device_id_type=pl.DeviceIdType.MESH,
            )
        pl.semaphore_wait(barrier_sem, 2)

        def rdma(src, dst, sem, dev):
            return pltpu.make_async_remote_copy(
                src_ref=src, dst_ref=dst,
                send_sem=send_sems.at[sem], recv_sem=recv_sems.at[sem],
                device_id=(dev,), device_id_type=pl.DeviceIdType.MESH,
            )

        xb = x_ref[...].astype(jnp.bfloat16)
        xb_ref[...] = xb
        xh0R = rdma(xb_ref, xw_ref.at[2], 0, right)
        xh0L = rdma(xb_ref, xw_ref.at[0], 1, left)
        xh0R.start()
        xh0L.start()

        scores = jnp.dot(xb, rw_ref[...], preferred_element_type=jnp.float32)
        p = jnp.exp(scores - jnp.max(scores, axis=-1, keepdims=True))
        p = p / jnp.sum(p, axis=-1, keepdims=True)
        eids = lax.broadcasted_iota(jnp.int32, (n_tok, E_TOTAL), 1).astype(
            jnp.float32
        )
        i0 = ridx_ref[:, 0:1]
        i1 = ridx_ref[:, 1:2]
        g0 = jnp.sum(jnp.where(eids == i0, p, 0.0), axis=-1, keepdims=True)
        g1 = jnp.sum(jnp.where(eids == i1, p, 0.0), axis=-1, keepdims=True)
        inv = 1.0 / (g0 + g1)
        auxs_ref[...] = jnp.concatenate(
            [i0, i1, g0 * inv, g1 * inv], axis=1
        ).astype(jnp.bfloat16)
        ah0R = rdma(auxs_ref, aw_ref.at[2], 2, right)
        ah0L = rdma(auxs_ref, aw_ref.at[0], 3, left)
        ah0R.start()
        ah0L.start()

        ew_cp0.wait()
        ew_cp1 = pltpu.make_async_copy(
            ew_ref.at[pl.ds(2, 2)], ewf_ref.at[pl.ds(2, 2)], ew_sem
        )
        ew_cp1.start()
        ewb_ref[pl.ds(0, 2)] = ewf_ref[pl.ds(0, 2)].astype(jnp.bfloat16)
        ew_cp1.wait()
        ewb_ref[pl.ds(2, 2)] = ewf_ref[pl.ds(2, 2)].astype(jnp.bfloat16)

        def partial(slot, c0, cw):
            if slot < 0:
                xc = xb
                ax = auxs_ref
            else:
                xc = xw_ref[slot]
                ax = aw_ref.at[slot]
            a_i0 = ax[:, 0:1].astype(jnp.float32)
            a_i1 = ax[:, 1:2].astype(jnp.float32)
            a_g0 = ax[:, 2:3].astype(jnp.float32)
            a_g1 = ax[:, 3:4].astype(jnp.float32)
            acc = jnp.zeros((n_tok, cw), jnp.float32)
            for j in range(E_LOC):
                ej = (E_LOC * my + j).astype(jnp.float32)
                gate = jnp.where(a_i0 == ej, a_g0, 0.0) + jnp.where(
                    a_i1 == ej, a_g1, 0.0
                )
                y = jnp.dot(
                    xc, ewb_ref[j, :, c0 : c0 + cw],
                    preferred_element_type=jnp.float32,
                )
                acc = acc + gate * y
            return acc

        p_me_L = partial(-1, 0, h2)
        p_me_R = partial(-1, h2, h2)

        xh0R.wait_recv()
        ah0R.wait_recv()
        xh1R = rdma(
            xw_ref.at[2, :, pl.ds(0, f2)], xw_ref.at[1, :, pl.ds(0, f2)],
            4, right,
        )
        ah1R = rdma(aw_ref.at[2], aw_ref.at[1], 5, right)
        xh1R.start()
        ah1R.start()
        xh0L.wait_recv()
        ah0L.wait_recv()
        xh1L = rdma(
            xw_ref.at[0, :, pl.ds(f2, f2)], xw_ref.at[1, :, pl.ds(f2, f2)],
            6, left,
        )
        xh1L.start()

        stR_ref[:, 0:q] = partial(2, 0, q).astype(jnp.bfloat16)
        rsR0a = rdma(stR_ref.at[:, pl.ds(0, q)], rbR_ref.at[0, :, pl.ds(0, q)], 7, right)
        rsR0a.start()
        stL_ref[:, 0:q] = partial(0, h2, q).astype(jnp.bfloat16)
        rsL0a = rdma(stL_ref.at[:, pl.ds(0, q)], rbL_ref.at[0, :, pl.ds(0, q)], 8, left)
        rsL0a.start()
        stR_ref[:, q:h2] = partial(2, q, q).astype(jnp.bfloat16)
        rsR0b = rdma(stR_ref.at[:, pl.ds(q, q)], rbR_ref.at[0, :, pl.ds(q, q)], 9, right)
        rsR0b.start()
        stL_ref[:, q:h2] = partial(0, h2 + q, q).astype(jnp.bfloat16)
        rsL0b = rdma(stL_ref.at[:, pl.ds(q, q)], rbL_ref.at[0, :, pl.ds(q, q)], 10, left)
        rsL0b.start()

        xh1R.wait_recv()
        xh1L.wait_recv()
        ah1R.wait_recv()

        p_dL0 = partial(1, 0, q).astype(jnp.bfloat16)
        rsR0a.wait_recv()
        rbR_ref[0, :, 0:q] = rbR_ref[0, :, 0:q] + p_dL0
        rsR1a = rdma(rbR_ref.at[0, :, pl.ds(0, q)], rbR_ref.at[1, :, pl.ds(0, q)], 11, right)
        rsR1a.start()

        p_dR0 = partial(1, h2, q).astype(jnp.bfloat16)
        rsL0a.wait_recv()
        rbL_ref[0, :, 0:q] = rbL_ref[0, :, 0:q] + p_dR0
        rsL1a = rdma(rbL_ref.at[0, :, pl.ds(0, q)], rbL_ref.at[1, :, pl.ds(0, q)], 12, left)
        rsL1a.start()

        p_dL1 = partial(1, q, q).astype(jnp.bfloat16)
        rsR0b.wait_recv()
        rbR_ref[0, :, q:h2] = rbR_ref[0, :, q:h2] + p_dL1
        rsR1b = rdma(rbR_ref.at[0, :, pl.ds(q, q)], rbR_ref.at[1, :, pl.ds(q, q)], 13, right)
        rsR1b.start()

        p_dR1 = partial(1, h2 + q, q).astype(jnp.bfloat16)
        rsL0b.wait_recv()
        rbL_ref[0, :, q:h2] = rbL_ref[0, :, q:h2] + p_dR1
        rsL1b = rdma(rbL_ref.at[0, :, pl.ds(q, q)], rbL_ref.at[1, :, pl.ds(q, q)], 14, left)
        rsL1b.start()

        p_nL0 = partial(0, 0, q).astype(jnp.bfloat16)
        rsR1a.wait_recv()
        rbR_ref[1, :, 0:q] = rbR_ref[1, :, 0:q] + p_nL0
        rsR2a = rdma(rbR_ref.at[1, :, pl.ds(0, q)], rbR_ref.at[2, :, pl.ds(0, q)], 15, right)
        rsR2a.start()

        p_nR0 = partial(2, h2, q).astype(jnp.bfloat16)
        rsL1a.wait_recv()
        rbL_ref[1, :, 0:q] = rbL_ref[1, :, 0:q] + p_nR0
        rsL2a = rdma(rbL_ref.at[1, :, pl.ds(0, q)], rbL_ref.at[2, :, pl.ds(0, q)], 16, left)
        rsL2a.start()

        p_nL1 = partial(0, q, q).astype(jnp.bfloat16)
        rsR1b.wait_recv()
        rbR_ref[1, :, q:h2] = rbR_ref[1, :, q:h2] + p_nL1
        rsR2b = rdma(rbR_ref.at[1, :, pl.ds(q, q)], rbR_ref.at[2, :, pl.ds(q, q)], 17, right)
        rsR2b.start()

        p_nR1 = partial(2, h2 + q, q).astype(jnp.bfloat16)
        rsL1b.wait_recv()
        rbL_ref[1, :, q:h2] = rbL_ref[1, :, q:h2] + p_nR1
        rsL2b = rdma(rbL_ref.at[1, :, pl.ds(q, q)], rbL_ref.at[2, :, pl.ds(q, q)], 18, left)
        rsL2b.start()

        def finish(idx, rb_ref, rbq0, outq0, p_me):
            outv_ref[:, outq0 : outq0 + q] = (
                rb_ref[2, :, rbq0 : rbq0 + q].astype(jnp.float32)
                + p_me[:, rbq0 : rbq0 + q]
            ).astype(jnp.bfloat16)
            cp = pltpu.make_async_copy(
                outv_ref.at[:, pl.ds(outq0, q)],
                out_ref.at[:, pl.ds(outq0, q)],
                out_sems.at[idx],
            )
            cp.start()
            return cp

        rsR2a.wait_recv()
        c0 = finish(0, rbR_ref, 0, 0, p_me_L)
        rsL2a.wait_recv()
        c1 = finish(1, rbL_ref, 0, h2, p_me_R)
        rsR2b.wait_recv()
        c2 = finish(2, rbR_ref, q, q, p_me_L)
        rsL2b.wait_recv()
        c3 = finish(3, rbL_ref, q, h2 + q, p_me_R)
        c0.wait()
        c1.wait()
        c2.wait()
        c3.wait()

        for d in (
            xh0R, xh0L, ah0R, ah0L, xh1R, ah1R, xh1L,
            rsR0a, rsL0a, rsR0b, rsL0b,
            rsR1a, rsL1a, rsR1b, rsL1b,
            rsR2a, rsL2a, rsR2b, rsL2b,
        ):
            d.wait_send()

    return pl.pallas_call(
        body,
        out_shape=jax.ShapeDtypeStruct((n_tok, d_hid), jnp.bfloat16),
        in_specs=[
            pl.BlockSpec(memory_space=pltpu.VMEM),
            pl.BlockSpec(memory_space=pltpu.VMEM),
            pl.BlockSpec(memory_space=pltpu.VMEM),
            pl.BlockSpec(memory_space=pl.ANY),
        ],
        out_specs=pl.BlockSpec(memory_space=pl.ANY),
        scratch_shapes=[
            pltpu.VMEM((n_tok, d_model), jnp.bfloat16),
            pltpu.VMEM((n_tok, 4), jnp.bfloat16),
            pltpu.VMEM((e_loc, d_model, d_hid), jnp.float32),
            pltpu.VMEM((e_loc, d_model, d_hid), jnp.bfloat16),
            pltpu.VMEM((3, n_tok, d_model), jnp.bfloat16),
            pltpu.VMEM((3, n_tok, 4), jnp.bfloat16),
            pltpu.VMEM((n_tok, h2), jnp.bfloat16),
            pltpu.VMEM((n_tok, h2), jnp.bfloat16),
            pltpu.VMEM((3, n_tok, h2), jnp.bfloat16),
            pltpu.VMEM((3, n_tok, h2), jnp.bfloat16),
            pltpu.VMEM((n_tok, d_hid), jnp.bfloat16),
            pltpu.SemaphoreType.DMA,
            pltpu.SemaphoreType.DMA((4,)),
            pltpu.SemaphoreType.DMA((19,)),
            pltpu.SemaphoreType.DMA((19,)),
        ],
        compiler_params=pltpu.CompilerParams(
            collective_id=0, vmem_limit_bytes=64 * 1024 * 1024
        ),
    )(x, rw_bf, ridx_f, expert_W)


# device time: 70936 ns/iter; 2.4517x vs baseline; 1.0184x over previous
import jax
import jax.numpy as jnp
from jax import lax
from jax.experimental import pallas as pl
from jax.experimental.pallas import tpu as pltpu

N_DEV = 4
E_TOTAL = 16
E_LOC = 4


def kernel(x, router_W, route_idx, expert_W):
    n_tok, d_model = x.shape
    e_loc, _, d_hid = expert_W.shape
    h2 = d_hid // 2
    q = d_hid // 4
    f2 = d_model // 2

    ridx_f = route_idx.astype(jnp.float32)
    rw_bf = router_W.astype(jnp.bfloat16)

    def body(
        x_ref, rw_ref, ridx_ref, ew_ref, out_ref,
        xb_ref,
        auxs_ref,
        ewf_ref,
        ewb_ref,
        xw_ref,
        aw_ref,
        stR_ref,
        stL_ref,
        rbR_ref,
        rbL_ref,
        outv_ref,
        ew_sem, out_sems, send_sems, recv_sems,
    ):
        my = lax.axis_index("i")
        left = lax.rem(my + N_DEV - 1, N_DEV)
        right = lax.rem(my + 1, N_DEV)

        ew_cp0 = pltpu.make_async_copy(
            ew_ref.at[pl.ds(0, 2)], ewf_ref.at[pl.ds(0, 2)], ew_sem
        )
        ew_cp0.start()

        barrier_sem = pltpu.get_barrier_semaphore()
        for nbr in (left, right):
            pl.semaphore_signal(
                barrier_sem, inc=1,
                device_id=(nbr,), device_id_type=pl.DeviceIdType.MESH,
            )
        pl.semaphore_wait(barrier_sem, 2)

        def rdma(src, dst, sem, dev):
            return pltpu.make_async_remote_copy(
                src_ref=src, dst_ref=dst,
                send_sem=send_sems.at[sem], recv_sem=recv_sems.at[sem],
                device_id=(dev,), device_id_type=pl.DeviceIdType.MESH,
            )

        t2 = n_tok // 2
        xb = x_ref[...].astype(jnp.bfloat16)
        xb_ref[...] = xb
        xh0Ra = rdma(
            xb_ref.at[pl.ds(0, t2)], xw_ref.at[2, pl.ds(0, t2), :], 0, right
        )
        xh0La = rdma(
            xb_ref.at[pl.ds(t2, t2)], xw_ref.at[0, pl.ds(t2, t2), :], 1, left
        )
        xh0Rb = rdma(
            xb_ref.at[pl.ds(t2, t2)], xw_ref.at[2, pl.ds(t2, t2), :], 19, right
        )
        xh0Lb = rdma(
            xb_ref.at[pl.ds(0, t2)], xw_ref.at[0, pl.ds(0, t2), :], 20, left
        )
        xh0Ra.start()
        xh0La.start()
        xh0Rb.start()
        xh0Lb.start()

        scores = jnp.dot(xb, rw_ref[...], preferred_element_type=jnp.float32)
        p = jnp.exp(scores - jnp.max(scores, axis=-1, keepdims=True))
        p = p / jnp.sum(p, axis=-1, keepdims=True)
        eids = lax.broadcasted_iota(jnp.int32, (n_tok, E_TOTAL), 1).astype(
            jnp.float32
        )
        i0 = ridx_ref[:, 0:1]
        i1 = ridx_ref[:, 1:2]
        g0 = jnp.sum(jnp.where(eids == i0, p, 0.0), axis=-1, keepdims=True)
        g1 = jnp.sum(jnp.where(eids == i1, p, 0.0), axis=-1, keepdims=True)
        inv = 1.0 / (g0 + g1)
        auxs_ref[...] = jnp.concatenate(
            [i0, i1, g0 * inv, g1 * inv], axis=1
        ).astype(jnp.bfloat16)
        ah0R = rdma(auxs_ref, aw_ref.at[2], 2, right)
        ah0L = rdma(auxs_ref, aw_ref.at[0], 3, left)
        ah0R.start()
        ah0L.start()

        ew_cp0.wait()
        ew_cp1 = pltpu.make_async_copy(
            ew_ref.at[pl.ds(2, 2)], ewf_ref.at[pl.ds(2, 2)], ew_sem
        )
        ew_cp1.start()
        ewb_ref[pl.ds(0, 2)] = ewf_ref[pl.ds(0, 2)].astype(jnp.bfloat16)
        ew_cp1.wait()
        ewb_ref[pl.ds(2, 2)] = ewf_ref[pl.ds(2, 2)].astype(jnp.bfloat16)

        def partial(slot, c0, cw):
            if slot < 0:
                xc = xb
                ax = auxs_ref
            else:
                xc = xw_ref[slot]
                ax = aw_ref.at[slot]
            a_i0 = ax[:, 0:1].astype(jnp.float32)
            a_i1 = ax[:, 1:2].astype(jnp.float32)
            a_g0 = ax[:, 2:3].astype(jnp.float32)
            a_g1 = ax[:, 3:4].astype(jnp.float32)
            acc = jnp.zeros((n_tok, cw), jnp.float32)
            for j in range(E_LOC):
                ej = (E_LOC * my + j).astype(jnp.float32)
                gate = jnp.where(a_i0 == ej, a_g0, 0.0) + jnp.where(
                    a_i1 == ej, a_g1, 0.0
                )
                y = jnp.dot(
                    xc, ewb_ref[j, :, c0 : c0 + cw],
                    preferred_element_type=jnp.float32,
                )
                acc = acc + gate * y
            return acc

        p_me_L = partial(-1, 0, h2)
        p_me_R = partial(-1, h2, h2)

        xh0Ra.wait_recv()
        xh1R = rdma(
            xw_ref.at[2, pl.ds(0, t2), :], xw_ref.at[1, pl.ds(0, t2), :],
            4, right,
        )
        xh1R.start()
        xh0La.wait_recv()
        xh1L = rdma(
            xw_ref.at[0, pl.ds(t2, t2), :], xw_ref.at[1, pl.ds(t2, t2), :],
            6, left,
        )
        xh1L.start()
        ah0R.wait_recv()
        ah1R = rdma(aw_ref.at[2], aw_ref.at[1], 5, right)
        ah1R.start()
        ah0L.wait_recv()
        xh0Rb.wait_recv()
        xh0Lb.wait_recv()

        stR_ref[:, 0:q] = partial(2, 0, q).astype(jnp.bfloat16)
        rsR0a = rdma(stR_ref.at[:, pl.ds(0, q)], rbR_ref.at[0, :, pl.ds(0, q)], 7, right)
        rsR0a.start()
        stL_ref[:, 0:q] = partial(0, h2, q).astype(jnp.bfloat16)
        rsL0a = rdma(stL_ref.at[:, pl.ds(0, q)], rbL_ref.at[0, :, pl.ds(0, q)], 8, left)
        rsL0a.start()
        stR_ref[:, q:h2] = partial(2, q, q).astype(jnp.bfloat16)
        rsR0b = rdma(stR_ref.at[:, pl.ds(q, q)], rbR_ref.at[0, :, pl.ds(q, q)], 9, right)
        rsR0b.start()
        stL_ref[:, q:h2] = partial(0, h2 + q, q).astype(jnp.bfloat16)
        rsL0b = rdma(stL_ref.at[:, pl.ds(q, q)], rbL_ref.at[0, :, pl.ds(q, q)], 10, left)
        rsL0b.start()

        xh1R.wait_recv()
        xh1L.wait_recv()
        ah1R.wait_recv()

        p_dL0 = partial(1, 0, q).astype(jnp.bfloat16)
        rsR0a.wait_recv()
        rbR_ref[0, :, 0:q] = rbR_ref[0, :, 0:q] + p_dL0
        rsR1a = rdma(rbR_ref.at[0, :, pl.ds(0, q)], rbR_ref.at[1, :, pl.ds(0, q)], 11, right)
        rsR1a.start()

        p_dR0 = partial(1, h2, q).astype(jnp.bfloat16)
        rsL0a.wait_recv()
        rbL_ref[0, :, 0:q] = rbL_ref[0, :, 0:q] + p_dR0
        rsL1a = rdma(rbL_ref.at[0, :, pl.ds(0, q)], rbL_ref.at[1, :, pl.ds(0, q)], 12, left)
        rsL1a.start()

        p_dL1 = partial(1, q, q).astype(jnp.bfloat16)
        rsR0b.wait_recv()
        rbR_ref[0, :, q:h2] = rbR_ref[0, :, q:h2] + p_dL1
        rsR1b = rdma(rbR_ref.at[0, :, pl.ds(q, q)], rbR_ref.at[1, :, pl.ds(q, q)], 13, right)
        rsR1b.start()

        p_dR1 = partial(1, h2 + q, q).astype(jnp.bfloat16)
        rsL0b.wait_recv()
        rbL_ref[0, :, q:h2] = rbL_ref[0, :, q:h2] + p_dR1
        rsL1b = rdma(rbL_ref.at[0, :, pl.ds(q, q)], rbL_ref.at[1, :, pl.ds(q, q)], 14, left)
        rsL1b.start()

        p_nL0 = partial(0, 0, q).astype(jnp.bfloat16)
        rsR1a.wait_recv()
        rbR_ref[1, :, 0:q] = rbR_ref[1, :, 0:q] + p_nL0
        rsR2a = rdma(rbR_ref.at[1, :, pl.ds(0, q)], rbR_ref.at[2, :, pl.ds(0, q)], 15, right)
        rsR2a.start()

        p_nR0 = partial(2, h2, q).astype(jnp.bfloat16)
        rsL1a.wait_recv()
        rbL_ref[1, :, 0:q] = rbL_ref[1, :, 0:q] + p_nR0
        rsL2a = rdma(rbL_ref.at[1, :, pl.ds(0, q)], rbL_ref.at[2, :, pl.ds(0, q)], 16, left)
        rsL2a.start()

        p_nL1 = partial(0, q, q).astype(jnp.bfloat16)
        rsR1b.wait_recv()
        rbR_ref[1, :, q:h2] = rbR_ref[1, :, q:h2] + p_nL1
        rsR2b = rdma(rbR_ref.at[1, :, pl.ds(q, q)], rbR_ref.at[2, :, pl.ds(q, q)], 17, right)
        rsR2b.start()

        p_nR1 = partial(2, h2 + q, q).astype(jnp.bfloat16)
        rsL1b.wait_recv()
        rbL_ref[1, :, q:h2] = rbL_ref[1, :, q:h2] + p_nR1
        rsL2b = rdma(rbL_ref.at[1, :, pl.ds(q, q)], rbL_ref.at[2, :, pl.ds(q, q)], 18, left)
        rsL2b.start()

        def finish(idx, rb_ref, rbq0, outq0, p_me):
            outv_ref[:, outq0 : outq0 + q] = (
                rb_ref[2, :, rbq0 : rbq0 + q].astype(jnp.float32)
                + p_me[:, rbq0 : rbq0 + q]
            ).astype(jnp.bfloat16)
            cp = pltpu.make_async_copy(
                outv_ref.at[:, pl.ds(outq0, q)],
                out_ref.at[:, pl.ds(outq0, q)],
                out_sems.at[idx],
            )
            cp.start()
            return cp

        rsR2a.wait_recv()
        c0 = finish(0, rbR_ref, 0, 0, p_me_L)
        rsL2a.wait_recv()
        c1 = finish(1, rbL_ref, 0, h2, p_me_R)
        rsR2b.wait_recv()
        c2 = finish(2, rbR_ref, q, q, p_me_L)
        rsL2b.wait_recv()
        c3 = finish(3, rbL_ref, q, h2 + q, p_me_R)
        c0.wait()
        c1.wait()
        c2.wait()
        c3.wait()

        for d in (
            xh0Ra, xh0La, xh0Rb, xh0Lb, ah0R, ah0L, xh1R, ah1R, xh1L,
            rsR0a, rsL0a, rsR0b, rsL0b,
            rsR1a, rsL1a, rsR1b, rsL1b,
            rsR2a, rsL2a, rsR2b, rsL2b,
        ):
            d.wait_send()

    return pl.pallas_call(
        body,
        out_shape=jax.ShapeDtypeStruct((n_tok, d_hid), jnp.bfloat16),
        in_specs=[
            pl.BlockSpec(memory_space=pltpu.VMEM),
            pl.BlockSpec(memory_space=pltpu.VMEM),
            pl.BlockSpec(memory_space=pltpu.VMEM),
            pl.BlockSpec(memory_space=pl.ANY),
        ],
        out_specs=pl.BlockSpec(memory_space=pl.ANY),
        scratch_shapes=[
            pltpu.VMEM((n_tok, d_model), jnp.bfloat16),
            pltpu.VMEM((n_tok, 4), jnp.bfloat16),
            pltpu.VMEM((e_loc, d_model, d_hid), jnp.float32),
            pltpu.VMEM((e_loc, d_model, d_hid), jnp.bfloat16),
            pltpu.VMEM((3, n_tok, d_model), jnp.bfloat16),
            pltpu.VMEM((3, n_tok, 4), jnp.bfloat16),
            pltpu.VMEM((n_tok, h2), jnp.bfloat16),
            pltpu.VMEM((n_tok, h2), jnp.bfloat16),
            pltpu.VMEM((3, n_tok, h2), jnp.bfloat16),
            pltpu.VMEM((3, n_tok, h2), jnp.bfloat16),
            pltpu.VMEM((n_tok, d_hid), jnp.bfloat16),
            pltpu.SemaphoreType.DMA,
            pltpu.SemaphoreType.DMA((4,)),
            pltpu.SemaphoreType.DMA((21,)),
            pltpu.SemaphoreType.DMA((21,)),
        ],
        compiler_params=pltpu.CompilerParams(
            collective_id=0, vmem_limit_bytes=64 * 1024 * 1024
        ),
    )(x, rw_bf, ridx_f, expert_W)
